# Initial kernel scaffold; baseline (speedup 1.0000x reference)
import jax
import jax.numpy as jnp
from jax.experimental import pallas as pl


def kernel(token_embeddings, indexes):
    raise NotImplementedError("write your pallas kernel here")


# speedup vs baseline: 1.3441x; 1.3441x over previous
import functools

import jax
import jax.numpy as jnp
from jax import lax
from jax.experimental import pallas as pl
from jax.experimental.pallas import tpu as pltpu
from jax.experimental.pallas import tpu_sc as plsc

B = 1024
T = 200
D = 768


def _build():
    info = plsc.get_sparse_core_info()
    NC, NS, L = info.num_cores, info.num_subcores, info.num_lanes
    NW = NC * NS
    bw = B // NW
    nrows = 2 * bw

    mesh = plsc.VectorSubcoreMesh(core_axis_name="c", subcore_axis_name="s")

    @functools.partial(
        pl.kernel,
        mesh=mesh,
        out_type=jax.ShapeDtypeStruct((2 * B, D), jnp.float32),
        scratch_types=[
            pltpu.VMEM((nrows,), jnp.int32),
            pltpu.VMEM((nrows,), jnp.int32),
            pltpu.VMEM((nrows,), jnp.int32),
            pltpu.VMEM((nrows, D), jnp.float32),
            pltpu.SemaphoreType.DMA,
            pltpu.SemaphoreType.DMA,
        ],
    )
    def k(table_hbm, idx_hbm, out_hbm, tok_v, flat_v, opos_v, rows_v, g_sem, s_sem):
        wid = lax.axis_index("s") * NC + lax.axis_index("c")
        wb = wid * bw
        pltpu.sync_copy(idx_hbm.at[pl.ds(wb, bw)], tok_v.at[pl.ds(0, bw)])
        pltpu.sync_copy(idx_hbm.at[pl.ds(B + wb, bw)], tok_v.at[pl.ds(bw, bw)])
        for i in range(nrows // L):
            j = i * L + lax.iota(jnp.int32, L)
            b = wb + lax.rem(j, bw)
            which = lax.div(j, bw)
            flat_v[pl.ds(i * L, L)] = b * T + tok_v[pl.ds(i * L, L)]
            opos_v[pl.ds(i * L, L)] = 2 * b + which
        gather = pltpu.async_copy(table_hbm.at[flat_v], rows_v, g_sem)
        gather.wait()
        pltpu.async_copy(rows_v, out_hbm.at[opos_v], s_sem).wait()

    return k


_gather_kernel = _build()


def kernel(token_embeddings, indexes):
    table = token_embeddings.reshape(B * T, D)
    idx = indexes.astype(jnp.int32).reshape(2 * B)
    out = _gather_kernel(table, idx)
    return out.reshape(B, 2 * D)


# speedup vs baseline: 1.3798x; 1.0266x over previous
import functools

import jax
import jax.numpy as jnp
from jax import lax
from jax.experimental import pallas as pl
from jax.experimental.pallas import tpu as pltpu
from jax.experimental.pallas import tpu_sc as plsc

B = 1024
T = 200
D = 768


def _build():
    info = plsc.get_sparse_core_info()
    NC, NS, L = info.num_cores, info.num_subcores, info.num_lanes
    NW = NC * NS
    nrows = 2 * B // NW

    mesh = plsc.VectorSubcoreMesh(core_axis_name="c", subcore_axis_name="s")

    @functools.partial(
        pl.kernel,
        mesh=mesh,
        out_type=jax.ShapeDtypeStruct((2 * B, D), jnp.float32),
        scratch_types=[
            pltpu.VMEM((nrows,), jnp.int32),
            pltpu.VMEM((nrows, D), jnp.float32),
            pltpu.SemaphoreType.DMA,
        ],
    )
    def k(table_hbm, idx_hbm, out_hbm, flat_v, rows_v, sem):
        wid = lax.axis_index("s") * NC + lax.axis_index("c")
        base = wid * nrows
        pltpu.sync_copy(idx_hbm.at[pl.ds(base, nrows)], flat_v)
        for i in range(nrows // L):
            j = base + i * L + lax.iota(jnp.int32, L)
            b = lax.shift_right_logical(j, 1)
            flat_v[pl.ds(i * L, L)] = b * T + flat_v[pl.ds(i * L, L)]
        pltpu.async_copy(table_hbm.at[flat_v], rows_v, sem).wait()
        pltpu.sync_copy(rows_v, out_hbm.at[pl.ds(base, nrows)])

    return k


_gather_kernel = _build()


def kernel(token_embeddings, indexes):
    table = token_embeddings.reshape(B * T, D)
    idx = jnp.swapaxes(indexes.astype(jnp.int32), 0, 1).reshape(2 * B)
    out = _gather_kernel(table, idx)
    return out.reshape(B, 2 * D)
